# Initial kernel scaffold; baseline (speedup 1.0000x reference)
import jax
import jax.numpy as jnp
from jax.experimental import pallas as pl


def kernel(site_probs, log_temperature):
    raise NotImplementedError("write your pallas kernel here")


# speedup vs baseline: 5.9093x; 5.9093x over previous
import jax
import jax.numpy as jnp
from jax.experimental import pallas as pl
from jax.experimental.pallas import tpu as pltpu

_TOPK = 20
_CAP = 1.0 - 1e-07


def _noisy_or_body(x_ref, lt_ref, o_ref):
    x = x_ref[...]
    xb = jax.lax.bitcast_convert_type(x, jnp.int32)
    rows = x.shape[0]

    def step(i, cand):
        bit = 30 - i
        trial = cand | (1 << bit)
        cnt = jnp.sum((xb >= trial).astype(jnp.int32), axis=1, keepdims=True)
        return jnp.where(cnt >= _TOPK, trial, cand)

    cand0 = jnp.zeros((rows, 1), jnp.int32)
    tb = jax.lax.fori_loop(0, 31, step, cand0)
    tf = jax.lax.bitcast_convert_type(tb, jnp.float32)

    inv_t = jnp.exp(-lt_ref[0])

    def log_survival(v):
        scaled = jnp.exp(jnp.log(v) * inv_t)
        return jnp.log1p(-jnp.minimum(scaled, _CAP))

    strict = xb > tb
    cnt_strict = jnp.sum(strict.astype(jnp.int32), axis=1, keepdims=True)
    s = jnp.sum(jnp.where(strict, log_survival(x), 0.0), axis=1, keepdims=True)
    s = s + (_TOPK - cnt_strict).astype(jnp.float32) * log_survival(tf)
    o_ref[...] = 1.0 - jnp.exp(s)


def kernel(site_probs, log_temperature):
    lt = jnp.reshape(log_temperature, (1,)).astype(jnp.float32)
    return pl.pallas_call(
        _noisy_or_body,
        out_shape=jax.ShapeDtypeStruct((site_probs.shape[0], 1), jnp.float32),
        in_specs=[
            pl.BlockSpec(memory_space=pltpu.VMEM),
            pl.BlockSpec(memory_space=pltpu.SMEM),
        ],
        out_specs=pl.BlockSpec(memory_space=pltpu.VMEM),
    )(site_probs, lt)
